# baseline (speedup 1.0000x reference)
import functools

import numpy as np
import jax
import jax.numpy as jnp
from jax import lax
from jax.experimental import pallas as pl
from jax.experimental.pallas import tpu as pltpu
from jax.experimental.pallas import tpu_sc as plsc

B, L, V = 1024, 50, 64
P = 0.05
N = B * L
PL, PV = 56, 128
NPAD = B * PL * PV
NC, NS = 2, 16
NW = NC * NS
ROWS_W = B // NW * PL
STEPS = 8
CHR = ROWS_W // STEPS
CW = CHR * PV
PAD = 16
SCALE_S = 1.0 / (V - 2)
SCALE_G = float(V - 1) / (V - 2)
PR = float(P / (V - 2))


def _shifted_mask_padded() -> np.ndarray:
    mask = np.random.RandomState(42).rand(N, V - 1) < P
    ash = np.zeros((B, PL, PV), np.float32)
    ash[:, :L, 1:V] = mask.reshape(B, L, V - 1)
    return ash.reshape(NPAD)


_ASH = _shifted_mask_padded()


def _sc_messages(m_pad, ash_pad):
    mesh = plsc.VectorSubcoreMesh(core_axis_name="c", subcore_axis_name="s")

    @functools.partial(
        pl.kernel,
        out_type=jax.ShapeDtypeStruct((NPAD,), jnp.float32),
        mesh=mesh,
        scratch_types=[
            pltpu.VMEM((PAD + CW,), jnp.float32),
            pltpu.VMEM((CW,), jnp.float32),
            pltpu.SemaphoreType.DMA,
        ],
        compiler_params=pltpu.CompilerParams(needs_layout_passes=False),
    )
    def k(m_hbm, ash_hbm, out_hbm, mbuf, abuf, sem):
        wid = lax.axis_index("s") * NC + lax.axis_index("c")
        base_w = wid * (ROWS_W * PV)
        mbuf[pl.ds(0, PAD)] = jnp.zeros((PAD,), jnp.float32)
        for step in range(STEPS):
            base = base_w + step * CW
            cm = pltpu.async_copy(m_hbm.at[pl.ds(base, CW)], mbuf.at[pl.ds(PAD, CW)], sem)
            ca = pltpu.async_copy(ash_hbm.at[pl.ds(base, CW)], abuf, sem)
            cm.wait()
            ca.wait()

            def batch_body(bi, carry):
                pr0 = bi * PL

                def row_body(l, carry2):
                    rb = (pr0 + l) * PV
                    gs = []
                    for kk in range(4):
                        mp = mbuf[pl.ds(PAD - 1 + rb + kk * 16, 16)]
                        av = abuf[pl.ds(rb + kk * 16, 16)]
                        gs.append(mp * av)
                    s = jnp.sum(gs[0] + gs[1] + gs[2] + gs[3]) * SCALE_S
                    sv = jnp.full((16,), s, jnp.float32)
                    sv0 = jnp.where(lax.iota(jnp.int32, 16) > 0, sv, 0.0)
                    for kk in range(4):
                        mm = mbuf[pl.ds(PAD + rb + kk * 16, 16)]
                        add = sv0 if kk == 0 else sv
                        mbuf[pl.ds(PAD + rb + kk * 16, 16)] = mm + add - SCALE_G * gs[kk]
                    return carry2

                lax.fori_loop(0, L, row_body, 0)
                return carry

            lax.fori_loop(0, CHR // PL, batch_body, 0)
            pltpu.sync_copy(mbuf.at[pl.ds(PAD, CW)], out_hbm.at[pl.ds(base, CW)])

    return k(m_pad, ash_pad)


def _tc_logits(l3d):
    BB = 64

    def body(l_ref, o_ref):
        l = l_ref[...]
        e = jnp.exp(l)
        e0 = e[:, :, 0:1]
        q = (1.0 - P) * e + PR * jnp.clip(1.0 - e - e0, 0.0, 1.0)
        col = lax.broadcasted_iota(jnp.int32, l.shape, 2)
        o_ref[...] = jnp.where(col == 0, l, jnp.log(q))

    return pl.pallas_call(
        body,
        grid=(B // BB,),
        in_specs=[pl.BlockSpec((BB, L, V), lambda i: (i, 0, 0))],
        out_specs=pl.BlockSpec((BB, L, V), lambda i: (i, 0, 0)),
        out_shape=jax.ShapeDtypeStruct((B, L, V), jnp.float32),
    )(l3d)


def kernel(messages, logits):
    ln = _tc_logits(logits)
    m_pad = jnp.pad(messages, ((0, 0), (0, PL - L), (0, PV - V))).reshape(NPAD)
    out_pad = _sc_messages(m_pad, jnp.asarray(_ASH))
    mn = out_pad.reshape(B, PL, PV)[:, :L, :V]
    return (mn, ln, messages, logits)


# speedup vs baseline: 1.0002x; 1.0002x over previous
import functools

import numpy as np
import jax
import jax.numpy as jnp
from jax import lax
from jax.experimental import pallas as pl
from jax.experimental.pallas import tpu as pltpu
from jax.experimental.pallas import tpu_sc as plsc

B, L, V = 1024, 50, 64
P = 0.05
N = B * L
PL, PV = 56, 128
NPAD = B * PL * PV
NC, NS = 2, 16
NW = NC * NS
ROWS_W = B // NW * PL
STEPS = 8
CHR = ROWS_W // STEPS
CW = CHR * PV
PAD = 16
SCALE_S = 1.0 / (V - 2)
SCALE_G = float(V - 1) / (V - 2)
PR = float(P / (V - 2))


def _shifted_mask_padded() -> np.ndarray:
    mask = np.random.RandomState(42).rand(N, V - 1) < P
    ash = np.zeros((B, PL, PV), np.float32)
    ash[:, :L, 1:V] = mask.reshape(B, L, V - 1)
    return ash.reshape(NPAD)


_ASH = _shifted_mask_padded()


def _sc_messages(m_pad, ash_pad):
    mesh = plsc.VectorSubcoreMesh(core_axis_name="c", subcore_axis_name="s")

    @functools.partial(
        pl.kernel,
        out_type=jax.ShapeDtypeStruct((NPAD,), jnp.float32),
        mesh=mesh,
        scratch_types=[
            pltpu.VMEM((PAD + CW,), jnp.float32),
            pltpu.VMEM((CW,), jnp.float32),
            pltpu.VMEM((CW,), jnp.float32),
            pltpu.SemaphoreType.DMA,
        ],
        compiler_params=pltpu.CompilerParams(needs_layout_passes=False),
    )
    def k(m_hbm, ash_hbm, out_hbm, mbuf, abuf, obuf, sem):
        wid = lax.axis_index("s") * NC + lax.axis_index("c")
        base_w = wid * (ROWS_W * PV)
        mbuf[pl.ds(0, PAD)] = jnp.zeros((PAD,), jnp.float32)
        for step in range(STEPS):
            base = base_w + step * CW
            cm = pltpu.async_copy(m_hbm.at[pl.ds(base, CW)], mbuf.at[pl.ds(PAD, CW)], sem)
            ca = pltpu.async_copy(ash_hbm.at[pl.ds(base, CW)], abuf, sem)
            cm.wait()
            ca.wait()

            def batch_body(bi, carry):
                pr0 = bi * PL

                def row_body(l, carry2):
                    rb = (pr0 + l) * PV
                    gs = []
                    for kk in range(4):
                        mp = mbuf[pl.ds(PAD - 1 + rb + kk * 16, 16)]
                        av = abuf[pl.ds(rb + kk * 16, 16)]
                        gs.append(mp * av)
                    s = jnp.sum(gs[0] + gs[1] + gs[2] + gs[3]) * SCALE_S
                    sv = jnp.full((16,), s, jnp.float32)
                    sv0 = jnp.where(lax.iota(jnp.int32, 16) > 0, sv, 0.0)
                    for kk in range(4):
                        mm = mbuf[pl.ds(PAD + rb + kk * 16, 16)]
                        add = sv0 if kk == 0 else sv
                        obuf[pl.ds(rb + kk * 16, 16)] = mm + add - SCALE_G * gs[kk]
                    return carry2

                lax.fori_loop(0, L, row_body, 0)
                return carry

            lax.fori_loop(0, CHR // PL, batch_body, 0)
            pltpu.sync_copy(obuf, out_hbm.at[pl.ds(base, CW)])

    return k(m_pad, ash_pad)


def _tc_logits(l3d):
    BB = 64

    def body(l_ref, o_ref):
        l = l_ref[...]
        e = jnp.exp(l)
        e0 = e[:, :, 0:1]
        q = (1.0 - P) * e + PR * jnp.clip(1.0 - e - e0, 0.0, 1.0)
        col = lax.broadcasted_iota(jnp.int32, l.shape, 2)
        o_ref[...] = jnp.where(col == 0, l, jnp.log(q))

    return pl.pallas_call(
        body,
        grid=(B // BB,),
        in_specs=[pl.BlockSpec((BB, L, V), lambda i: (i, 0, 0))],
        out_specs=pl.BlockSpec((BB, L, V), lambda i: (i, 0, 0)),
        out_shape=jax.ShapeDtypeStruct((B, L, V), jnp.float32),
    )(l3d)


def kernel(messages, logits):
    ln = _tc_logits(logits)
    m_pad = jnp.pad(messages, ((0, 0), (0, PL - L), (0, PV - V))).reshape(NPAD)
    out_pad = _sc_messages(m_pad, jnp.asarray(_ASH))
    mn = out_pad.reshape(B, PL, PV)[:, :L, :V]
    return (mn, ln, messages, logits)


# speedup vs baseline: 1.2829x; 1.2826x over previous
import functools

import numpy as np
import jax
import jax.numpy as jnp
from jax import lax
from jax.experimental import pallas as pl
from jax.experimental.pallas import tpu as pltpu
from jax.experimental.pallas import tpu_sc as plsc

B, L, V = 1024, 50, 64
P = 0.05
N = B * L
NC, NS = 2, 16
NW = NC * NS
ROWS_W = N // NW
CH = 200
STEPS = ROWS_W // CH
CW = CH * V
PAD = 16
SCALE_S = 1.0 / (V - 2)
SCALE_G = float(V - 1) / (V - 2)
PR = float(P / (V - 2))
MROWS = N * V // 128


def _shifted_mask() -> np.ndarray:
    mask = np.random.RandomState(42).rand(N, V - 1) < P
    a = np.zeros((N, V), np.float32)
    a[:, : V - 1] = mask
    ash_rows = np.concatenate([np.zeros((N, 1), np.float32), a[:, : V - 1]], axis=1)
    ar = ash_rows.reshape(B, L, V)
    return np.concatenate([ar[:, :25, :], ar[:, 25:, :]], axis=2).reshape(N * V)


_ASH = _shifted_mask()


def _sc_messages(m_flat, ash_flat):
    mesh = plsc.VectorSubcoreMesh(core_axis_name="c", subcore_axis_name="s")

    @functools.partial(
        pl.kernel,
        out_type=jax.ShapeDtypeStruct((N * V,), jnp.float32),
        mesh=mesh,
        scratch_types=[
            pltpu.VMEM((PAD + CW,), jnp.float32),
            pltpu.VMEM((CW,), jnp.float32),
            pltpu.VMEM((CW,), jnp.float32),
            pltpu.SemaphoreType.DMA,
        ],
        compiler_params=pltpu.CompilerParams(needs_layout_passes=False),
    )
    def k(m_hbm, ash_hbm, out_hbm, mbuf, abuf, obuf, sem):
        wid = lax.axis_index("s") * NC + lax.axis_index("c")
        base_w = wid * (ROWS_W * V)
        for step in range(STEPS):
            base = base_w + step * CW
            cm = pltpu.async_copy(m_hbm.at[pl.ds(base, CW)], mbuf.at[pl.ds(PAD, CW)], sem)
            ca = pltpu.async_copy(ash_hbm.at[pl.ds(base, CW)], abuf, sem)
            cm.wait()
            ca.wait()

            def row_body(r, carry):
                rb = r * V
                gs = []
                for kk in range(4):
                    mp = mbuf[pl.ds(PAD - 1 + rb + kk * 16, 16)]
                    av = abuf[pl.ds(rb + kk * 16, 16)]
                    gs.append(mp * av)
                s = jnp.sum(gs[0] + gs[1] + gs[2] + gs[3]) * SCALE_S
                sv = jnp.full((16,), s, jnp.float32)
                sv0 = jnp.where(lax.iota(jnp.int32, 16) > 0, sv, 0.0)
                for kk in range(4):
                    mm = mbuf[pl.ds(PAD + rb + kk * 16, 16)]
                    add = sv0 if kk == 0 else sv
                    obuf[pl.ds(rb + kk * 16, 16)] = mm + add - SCALE_G * gs[kk]
                return carry

            lax.fori_loop(0, CH, row_body, 0)
            pltpu.sync_copy(obuf, out_hbm.at[pl.ds(base, CW)])

    return k(m_flat, ash_flat)


BG = 4
GB = BG * 8


def _tc_pre(messages, logits):
    def body(m_ref, l_ref, f_ref, o_ref):
        for g in range(BG):
            for j in range(8):
                mj = m_ref[g * 8 + j]
                cat = jnp.concatenate([mj[:25, :], mj[25:, :]], axis=1)
                f_ref[g, pl.ds(j * 25, 25), :] = cat
        l = l_ref[...]
        e = jnp.exp(l)
        e0 = e[:, :, 0:1]
        q = (1.0 - P) * e + PR * jnp.clip(1.0 - e - e0, 0.0, 1.0)
        col = lax.broadcasted_iota(jnp.int32, l.shape, 2)
        o_ref[...] = jnp.where(col == 0, l, jnp.log(q))

    return pl.pallas_call(
        body,
        grid=(B // GB,),
        in_specs=[
            pl.BlockSpec((GB, L, V), lambda i: (i, 0, 0)),
            pl.BlockSpec((GB, L, V), lambda i: (i, 0, 0)),
        ],
        out_specs=[
            pl.BlockSpec((BG, 200, 128), lambda i: (i, 0, 0)),
            pl.BlockSpec((GB, L, V), lambda i: (i, 0, 0)),
        ],
        out_shape=[
            jax.ShapeDtypeStruct((B // 8, 200, 128), jnp.float32),
            jax.ShapeDtypeStruct((B, L, V), jnp.float32),
        ],
    )(messages, logits)


def _tc_post(out3d):
    def body(f_ref, o_ref):
        for g in range(BG):
            for j in range(8):
                c = f_ref[g, pl.ds(j * 25, 25), :]
                o_ref[g * 8 + j, pl.ds(0, 25), :] = c[:, :64]
                o_ref[g * 8 + j, pl.ds(25, 25), :] = c[:, 64:]

    return pl.pallas_call(
        body,
        grid=(B // GB,),
        in_specs=[pl.BlockSpec((BG, 200, 128), lambda i: (i, 0, 0))],
        out_specs=pl.BlockSpec((GB, L, V), lambda i: (i, 0, 0)),
        out_shape=jax.ShapeDtypeStruct((B, L, V), jnp.float32),
    )(out3d)


def kernel(messages, logits):
    m3d, ln = _tc_pre(messages, logits)
    out_flat = _sc_messages(m3d.reshape(N * V), jnp.asarray(_ASH))
    mn = _tc_post(out_flat.reshape(B // 8, 200, 128))
    return (mn, ln, messages, logits)


# speedup vs baseline: 2.3031x; 1.7952x over previous
import functools

import numpy as np
import jax
import jax.numpy as jnp
from jax import lax
from jax.experimental import pallas as pl
from jax.experimental.pallas import tpu as pltpu
from jax.experimental.pallas import tpu_sc as plsc

B, L, V = 1024, 50, 64
P = 0.05
N = B * L
R = L * V
NC, NS = 2, 16
NW = NC * NS
LB = 128
MAXL = 13
SCALE_S = 1.0 / (V - 2)
SCALE_G = float(V - 1) / (V - 2)
PR = float(P / (V - 2))


def _mask_t() -> np.ndarray:
    mask = np.random.RandomState(42).rand(N, V - 1) < P
    a = np.zeros((B, L, V), np.float32)
    a[:, :, : V - 1] = mask.reshape(B, L, V - 1)
    return np.ascontiguousarray(a.transpose(1, 2, 0)).reshape(R, B)


_AT = _mask_t()


def _sc_messages_t(m_t, a_t):
    mesh = plsc.VectorSubcoreMesh(core_axis_name="c", subcore_axis_name="s")

    @functools.partial(
        pl.kernel,
        out_type=jax.ShapeDtypeStruct((R, B), jnp.float32),
        mesh=mesh,
        scratch_types=[
            pltpu.VMEM((V, LB), jnp.float32),
            pltpu.VMEM((V, LB), jnp.float32),
            pltpu.VMEM((V, LB), jnp.float32),
            pltpu.SemaphoreType.DMA,
        ],
    )
    def k(m_hbm, a_hbm, out_hbm, mbuf, abuf, obuf, sem):
        wid = lax.axis_index("s") * NC + lax.axis_index("c")
        lg = wid // 8
        b0 = (wid % 8) * LB
        l_start = MAXL * lg - jnp.maximum(lg - 2, 0)
        l_len = MAXL - (lg >= 2).astype(jnp.int32)

        def step_body(st, carry):
            @pl.when(st < l_len)
            def _():
                r0 = (l_start + st) * V
                cm = pltpu.async_copy(
                    m_hbm.at[pl.ds(r0, V), pl.ds(b0, LB)], mbuf, sem)
                ca = pltpu.async_copy(
                    a_hbm.at[pl.ds(r0, V), pl.ds(b0, LB)], abuf, sem)
                cm.wait()
                ca.wait()

                def g_body(g, c2):
                    sl = pl.ds(g * 16, 16)
                    s = jnp.zeros((16,), jnp.float32)
                    for v in range(V):
                        s = s + mbuf[v, sl] * abuf[v, sl]
                    s = s * SCALE_S
                    gprev = None
                    for v in range(V):
                        mv = mbuf[v, sl]
                        if v == 0:
                            obuf[v, sl] = mv
                        else:
                            obuf[v, sl] = mv + s - SCALE_G * gprev
                        if v < V - 1:
                            gprev = mv * abuf[v, sl]
                    return c2

                lax.fori_loop(0, LB // 16, g_body, 0)
                pltpu.sync_copy(obuf, out_hbm.at[pl.ds(r0, V), pl.ds(b0, LB)])

            return carry

        lax.fori_loop(0, MAXL, step_body, 0)

    return k(m_t, a_t)


def _tc_logits_t(l_t):
    BB = 128

    def body(l_ref, o_ref):
        l = l_ref[...]
        e = jnp.exp(l)
        e0 = e[:, 0:1, :]
        q = (1.0 - P) * e + PR * jnp.clip(1.0 - e - e0, 0.0, 1.0)
        col = lax.broadcasted_iota(jnp.int32, l.shape, 1)
        o_ref[...] = jnp.where(col == 0, l, jnp.log(q))

    return pl.pallas_call(
        body,
        grid=(B // BB,),
        in_specs=[pl.BlockSpec((L, V, BB), lambda i: (0, 0, i))],
        out_specs=pl.BlockSpec((L, V, BB), lambda i: (0, 0, i)),
        out_shape=jax.ShapeDtypeStruct((L, V, B), jnp.float32),
    )(l_t)


def kernel(messages, logits):
    m_t = jnp.transpose(messages, (1, 2, 0))
    l_t = jnp.transpose(logits, (1, 2, 0))
    ln_t = _tc_logits_t(l_t)
    mn2d = _sc_messages_t(m_t.reshape(R, B), jnp.asarray(_AT))
    mn = jnp.transpose(mn2d.reshape(L, V, B), (2, 0, 1))
    ln = jnp.transpose(ln_t, (2, 0, 1))
    return (mn, ln, messages, logits)


# speedup vs baseline: 3.5347x; 1.5347x over previous
import functools

import numpy as np
import jax
import jax.numpy as jnp
from jax import lax
from jax.experimental import pallas as pl
from jax.experimental.pallas import tpu as pltpu
from jax.experimental.pallas import tpu_sc as plsc

B, L, V = 1024, 50, 64
P = 0.05
N = B * L
R = L * V
NC, NS = 2, 16
NW = NC * NS
LB = 128
MAXL = 13
SCALE_S = 1.0 / (V - 2)
SCALE_G = float(V - 1) / (V - 2)
PR = float(P / (V - 2))


def _mask_t() -> np.ndarray:
    mask = np.random.RandomState(42).rand(N, V - 1) < P
    a = np.zeros((B, L, V), np.float32)
    a[:, :, : V - 1] = mask.reshape(B, L, V - 1)
    at = np.ascontiguousarray(a.transpose(1, 2, 0)).reshape(L, V, B)
    pair = np.stack([at[:, :32, :], at[:, 32:, :]], axis=-1)
    bf = np.asarray(pair, dtype=jnp.bfloat16)
    return np.ascontiguousarray(bf).view(np.int32).reshape(L * 32, B)


_AT = _mask_t()


def _sc_messages_t(m_t, a_t):
    mesh = plsc.VectorSubcoreMesh(core_axis_name="c", subcore_axis_name="s")

    def amask(abuf, vv, sl):
        ab = plsc.bitcast(abuf[vv, sl], jnp.bfloat16)
        return plsc.unpack(ab, format=plsc.PackFormat.INTERLEAVED)

    def compute(mbuf, abuf, obuf):
        def group_body(g, carry):
            sl = pl.ds(g * 16, 16)
            ghi_init, _ = amask(abuf, 31, sl)
            ghi = mbuf[31, sl] * ghi_init
            glo = None
            s = jnp.zeros((16,), jnp.float32)
            for vv in range(32):
                mlo = mbuf[vv, sl]
                mhi = mbuf[vv + 32, sl]
                if vv == 0:
                    obuf[vv, sl] = mlo
                else:
                    obuf[vv, sl] = mlo - SCALE_G * glo
                obuf[vv + 32, sl] = mhi - SCALE_G * ghi
                alo, ahi = amask(abuf, vv, sl)
                glo = mlo * alo
                ghi = mhi * ahi
                s = s + glo + ghi
            sv = s * SCALE_S
            for v in range(1, V):
                plsc.addupdate(obuf.at[v, sl], sv)
            return carry

        lax.fori_loop(0, LB // 16, group_body, 0)

    @functools.partial(
        pl.kernel,
        out_type=jax.ShapeDtypeStruct((R, B), jnp.float32),
        mesh=mesh,
        scratch_types=[
            pltpu.VMEM((V, LB), jnp.float32),
            pltpu.VMEM((V, LB), jnp.float32),
            pltpu.VMEM((32, LB), jnp.int32),
            pltpu.VMEM((32, LB), jnp.int32),
            pltpu.VMEM((V, LB), jnp.float32),
            pltpu.VMEM((V, LB), jnp.float32),
            pltpu.SemaphoreType.DMA,
            pltpu.SemaphoreType.DMA,
        ],
        compiler_params=pltpu.CompilerParams(needs_layout_passes=False),
    )
    def k(m_hbm, a_hbm, out_hbm, mb0, mb1, ab0, ab1, ob0, ob1, semA, semB):
        wid = lax.axis_index("s") * NC + lax.axis_index("c")
        lg = wid // 8
        b0 = (wid % 8) * LB
        l_start = MAXL * lg - jnp.maximum(lg - 2, 0)
        l_len = MAXL - (lg >= 2).astype(jnp.int32)

        def issue(st, mb, ab, sem):
            l = l_start + st
            pltpu.async_copy(m_hbm.at[pl.ds(l * V, V), pl.ds(b0, LB)], mb, sem)
            pltpu.async_copy(a_hbm.at[pl.ds(l * 32, 32), pl.ds(b0, LB)], ab, sem)

        def drain(st, mb, ab, sem):
            l = l_start + st
            pltpu.make_async_copy(
                m_hbm.at[pl.ds(l * V, V), pl.ds(b0, LB)], mb, sem).wait()
            pltpu.make_async_copy(
                a_hbm.at[pl.ds(l * 32, 32), pl.ds(b0, LB)], ab, sem).wait()

        def store_out(st, ob):
            l = l_start + st
            pltpu.sync_copy(ob, out_hbm.at[pl.ds(l * V, V), pl.ds(b0, LB)])

        issue(0, mb0, ab0, semA)

        def t_body(t, carry):
            st0 = 2 * t
            st1 = st0 + 1

            @pl.when(st1 < l_len)
            def _():
                issue(st1, mb1, ab1, semB)

            @pl.when(st0 < l_len)
            def _():
                drain(st0, mb0, ab0, semA)
                compute(mb0, ab0, ob0)
                store_out(st0, ob0)

            @pl.when(st0 + 2 < l_len)
            def _():
                issue(st0 + 2, mb0, ab0, semA)

            @pl.when(st1 < l_len)
            def _():
                drain(st1, mb1, ab1, semB)
                compute(mb1, ab1, ob1)
                store_out(st1, ob1)

            return carry

        lax.fori_loop(0, (MAXL + 1) // 2, t_body, 0)

    return k(m_t, a_t)


def _tc_logits_t(l_t):
    BB = 128

    def body(l_ref, o_ref):
        l = l_ref[...]
        e = jnp.exp(l)
        e0 = e[:, 0:1, :]
        q = (1.0 - P) * e + PR * jnp.clip(1.0 - e - e0, 0.0, 1.0)
        col = lax.broadcasted_iota(jnp.int32, l.shape, 1)
        o_ref[...] = jnp.where(col == 0, l, jnp.log(q))

    return pl.pallas_call(
        body,
        grid=(B // BB,),
        in_specs=[pl.BlockSpec((L, V, BB), lambda i: (0, 0, i))],
        out_specs=pl.BlockSpec((L, V, BB), lambda i: (0, 0, i)),
        out_shape=jax.ShapeDtypeStruct((L, V, B), jnp.float32),
    )(l_t)


def kernel(messages, logits):
    m_t = jnp.transpose(messages, (1, 2, 0))
    l_t = jnp.transpose(logits, (1, 2, 0))
    ln_t = _tc_logits_t(l_t)
    mn2d = _sc_messages_t(m_t.reshape(R, B), jnp.asarray(_AT))
    mn = jnp.transpose(mn2d.reshape(L, V, B), (2, 0, 1))
    ln = jnp.transpose(ln_t, (2, 0, 1))
    return (mn, ln, messages, logits)


# speedup vs baseline: 4.7198x; 1.3353x over previous
import functools

import numpy as np
import jax
import jax.numpy as jnp
from jax import lax
from jax.experimental import pallas as pl
from jax.experimental.pallas import tpu as pltpu
from jax.experimental.pallas import tpu_sc as plsc

B, L, V = 1024, 50, 64
P = 0.05
N = B * L
R = L * V
NC, NS = 2, 16
NW = NC * NS
LB = 128
MAXL = 13
SCALE_S = 1.0 / (V - 2)
SCALE_G = float(V - 1) / (V - 2)
PR = float(P / (V - 2))


def _mask_t() -> np.ndarray:
    mask = np.random.RandomState(42).rand(N, V - 1) < P
    a = np.zeros((B, L, V), np.float32)
    a[:, :, : V - 1] = mask.reshape(B, L, V - 1)
    at = np.ascontiguousarray(a.transpose(1, 2, 0)).reshape(L, V, B)
    pair = np.stack([at[:, :32, :], at[:, 32:, :]], axis=-1)
    bf = np.asarray(pair, dtype=jnp.bfloat16)
    return np.ascontiguousarray(bf).view(np.int32).reshape(L * 32, B)


_AT = _mask_t()


def _sc_messages_t(m_t, a_t):
    mesh = plsc.VectorSubcoreMesh(core_axis_name="c", subcore_axis_name="s")

    def amask(abuf, vv, sl):
        ab = plsc.bitcast(abuf[vv, sl], jnp.bfloat16)
        return plsc.unpack(ab, format=plsc.PackFormat.INTERLEAVED)

    def compute(mbuf, abuf, obuf):
        def group_body(g, carry):
            sl = pl.ds(g * 16, 16)
            ghi_init, _ = amask(abuf, 31, sl)
            ghi = mbuf[31, sl] * ghi_init
            glo = None
            s = jnp.zeros((16,), jnp.float32)
            for vv in range(32):
                mlo = mbuf[vv, sl]
                mhi = mbuf[vv + 32, sl]
                if vv == 0:
                    obuf[vv, sl] = mlo
                else:
                    obuf[vv, sl] = mlo - SCALE_G * glo
                obuf[vv + 32, sl] = mhi - SCALE_G * ghi
                alo, ahi = amask(abuf, vv, sl)
                glo = mlo * alo
                ghi = mhi * ahi
                s = s + glo + ghi
            sv = s * SCALE_S
            for v in range(1, V):
                plsc.addupdate(obuf.at[v, sl], sv)
            return carry

        lax.fori_loop(0, LB // 16, group_body, 0)

    @functools.partial(
        pl.kernel,
        out_type=jax.ShapeDtypeStruct((R, B), jnp.float32),
        mesh=mesh,
        scratch_types=[
            pltpu.VMEM((V, LB), jnp.float32),
            pltpu.VMEM((V, LB), jnp.float32),
            pltpu.VMEM((32, LB), jnp.int32),
            pltpu.VMEM((32, LB), jnp.int32),
            pltpu.VMEM((V, LB), jnp.float32),
            pltpu.VMEM((V, LB), jnp.float32),
            pltpu.SemaphoreType.DMA,
            pltpu.SemaphoreType.DMA,
        ],
        compiler_params=pltpu.CompilerParams(needs_layout_passes=False),
    )
    def k(m_hbm, a_hbm, out_hbm, mb0, mb1, ab0, ab1, ob0, ob1, semA, semB):
        wid = lax.axis_index("s") * NC + lax.axis_index("c")
        lg = wid // 8
        b0 = (wid % 8) * LB
        l_start = MAXL * lg - jnp.maximum(lg - 2, 0)
        l_len = MAXL - (lg >= 2).astype(jnp.int32)

        def issue(st, mb, ab, sem):
            l = l_start + st
            pltpu.async_copy(m_hbm.at[pl.ds(l * V, V), pl.ds(b0, LB)], mb, sem)
            pltpu.async_copy(a_hbm.at[pl.ds(l * 32, 32), pl.ds(b0, LB)], ab, sem)

        def drain(st, mb, ab, sem):
            l = l_start + st
            pltpu.make_async_copy(
                m_hbm.at[pl.ds(l * V, V), pl.ds(b0, LB)], mb, sem).wait()
            pltpu.make_async_copy(
                a_hbm.at[pl.ds(l * 32, 32), pl.ds(b0, LB)], ab, sem).wait()

        def store_out(st, ob):
            l = l_start + st
            pltpu.sync_copy(ob, out_hbm.at[pl.ds(l * V, V), pl.ds(b0, LB)])

        issue(0, mb0, ab0, semA)

        def t_body(t, carry):
            st0 = 2 * t
            st1 = st0 + 1

            @pl.when(st1 < l_len)
            def _():
                issue(st1, mb1, ab1, semB)

            @pl.when(st0 < l_len)
            def _():
                drain(st0, mb0, ab0, semA)
                compute(mb0, ab0, ob0)
                store_out(st0, ob0)

            @pl.when(st0 + 2 < l_len)
            def _():
                issue(st0 + 2, mb0, ab0, semA)

            @pl.when(st1 < l_len)
            def _():
                drain(st1, mb1, ab1, semB)
                compute(mb1, ab1, ob1)
                store_out(st1, ob1)

            return carry

        lax.fori_loop(0, (MAXL + 1) // 2, t_body, 0)

    return k(m_t, a_t)


def _tc_logits_t(m_t, l_t):
    BB = 128

    def body(m_ref, l_ref, o_ref, om_ref, ol_ref):
        l = l_ref[...]
        e = jnp.exp(l)
        e0 = e[:, 0:1, :]
        q = (1.0 - P) * e + PR * jnp.clip(1.0 - e - e0, 0.0, 1.0)
        col = lax.broadcasted_iota(jnp.int32, l.shape, 1)
        o_ref[...] = jnp.where(col == 0, l, jnp.log(q))
        om_ref[...] = m_ref[...]
        ol_ref[...] = l

    spec = pl.BlockSpec((L, V, BB), lambda i: (0, 0, i))
    return pl.pallas_call(
        body,
        grid=(B // BB,),
        in_specs=[spec, spec],
        out_specs=[spec, spec, spec],
        out_shape=[jax.ShapeDtypeStruct((L, V, B), jnp.float32)] * 3,
    )(m_t, l_t)


def kernel(messages, logits):
    m_t = jnp.transpose(messages, (1, 2, 0))
    l_t = jnp.transpose(logits, (1, 2, 0))
    ln_t, mcp_t, lcp_t = _tc_logits_t(m_t, l_t)
    mn2d = _sc_messages_t(m_t.reshape(R, B), jnp.asarray(_AT))
    mn = jnp.transpose(mn2d.reshape(L, V, B), (2, 0, 1))
    ln = jnp.transpose(ln_t, (2, 0, 1))
    return (mn, ln, jnp.transpose(mcp_t, (2, 0, 1)), jnp.transpose(lcp_t, (2, 0, 1)))


# speedup vs baseline: 5.6685x; 1.2010x over previous
import functools

import numpy as np
import jax
import jax.numpy as jnp
from jax import lax
from jax.experimental import pallas as pl
from jax.experimental.pallas import tpu as pltpu
from jax.experimental.pallas import tpu_sc as plsc

B, L, V = 1024, 50, 64
P = 0.05
N = B * L
R = L * V
NC, NS = 2, 16
NW = NC * NS
LB = 128
MAXL = 13
SCALE_S = 1.0 / (V - 2)
SCALE_G = float(V - 1) / (V - 2)
PR = float(P / (V - 2))


def _mask_t() -> np.ndarray:
    mask = np.random.RandomState(42).rand(N, V - 1) < P
    a = np.zeros((B, L, V), np.uint32)
    a[:, :, : V - 1] = mask.reshape(B, L, V - 1)
    at = np.ascontiguousarray(a.transpose(1, 2, 0))
    w = np.zeros((L, 2, B), np.uint32)
    for vv in range(32):
        w[:, 0, :] |= at[:, vv, :] << vv
        w[:, 1, :] |= at[:, vv + 32, :] << vv
    return w.reshape(L * 2, B).view(np.int32)


_AT = _mask_t()


def _sc_messages_t(m_t, a_t):
    mesh = plsc.VectorSubcoreMesh(core_axis_name="c", subcore_axis_name="s")

    def compute(l, mbuf, abuf, obuf):
        def group_body(g, carry):
            sl = pl.ds(g * 16, 16)
            one = jnp.ones((16,), jnp.int32)
            wlo = abuf[2 * l, sl]
            whi = abuf[2 * l + 1, sl]

            def bit(w, vv):
                return ((lax.shift_right_logical(w, jnp.full((16,), vv, jnp.int32)) & one)
                        .astype(jnp.float32))

            ghi = mbuf[31, sl] * bit(wlo, 31)
            glo = None
            s = jnp.zeros((16,), jnp.float32)
            for vv in range(32):
                mlo = mbuf[vv, sl]
                mhi = mbuf[vv + 32, sl]
                if vv == 0:
                    obuf[vv, sl] = mlo
                else:
                    obuf[vv, sl] = mlo - SCALE_G * glo
                obuf[vv + 32, sl] = mhi - SCALE_G * ghi
                glo = mlo * bit(wlo, vv)
                ghi = mhi * bit(whi, vv)
                s = s + glo + ghi
            sv = s * SCALE_S
            for v in range(1, V):
                plsc.addupdate(obuf.at[v, sl], sv)
            return carry

        lax.fori_loop(0, LB // 16, group_body, 0)

    @functools.partial(
        pl.kernel,
        out_type=jax.ShapeDtypeStruct((R, B), jnp.float32),
        mesh=mesh,
        scratch_types=[
            pltpu.VMEM((V, LB), jnp.float32),
            pltpu.VMEM((V, LB), jnp.float32),
            pltpu.VMEM((2 * L, LB), jnp.int32),
            pltpu.VMEM((V, LB), jnp.float32),
            pltpu.VMEM((V, LB), jnp.float32),
            pltpu.SemaphoreType.DMA,
            pltpu.SemaphoreType.DMA,
            pltpu.SemaphoreType.DMA,
            pltpu.SemaphoreType.DMA,
        ],
        compiler_params=pltpu.CompilerParams(needs_layout_passes=False),
    )
    def k(m_hbm, a_hbm, out_hbm, mb0, mb1, abuf, ob0, ob1, semA, semB, semO0, semO1):
        wid = lax.axis_index("s") * NC + lax.axis_index("c")
        lg = wid // 8
        b0 = (wid % 8) * LB
        l_start = MAXL * lg - jnp.maximum(lg - 2, 0)
        l_len = MAXL - (lg >= 2).astype(jnp.int32)

        def issue(st, mb, sem):
            l = l_start + st
            pltpu.async_copy(m_hbm.at[pl.ds(l * V, V), pl.ds(b0, LB)], mb, sem)

        def drain(st, mb, sem):
            l = l_start + st
            pltpu.make_async_copy(
                m_hbm.at[pl.ds(l * V, V), pl.ds(b0, LB)], mb, sem).wait()

        def issue_out(st, ob, sem):
            l = l_start + st
            pltpu.async_copy(ob, out_hbm.at[pl.ds(l * V, V), pl.ds(b0, LB)], sem)

        def drain_out(st, ob, sem):
            l = l_start + st
            pltpu.make_async_copy(
                ob, out_hbm.at[pl.ds(l * V, V), pl.ds(b0, LB)], sem).wait()

        ca = pltpu.async_copy(a_hbm.at[:, pl.ds(b0, LB)], abuf, semA)
        issue(0, mb0, semA)
        ca.wait()

        def t_body(t, carry):
            st0 = 2 * t
            st1 = st0 + 1

            @pl.when(st1 < l_len)
            def _():
                issue(st1, mb1, semB)

            @pl.when(st0 < l_len)
            def _():
                drain(st0, mb0, semA)

                @pl.when(st0 >= 2)
                def _():
                    drain_out(st0 - 2, ob0, semO0)

                compute(l_start + st0, mb0, abuf, ob0)
                issue_out(st0, ob0, semO0)

            @pl.when(st0 + 2 < l_len)
            def _():
                issue(st0 + 2, mb0, semA)

            @pl.when(st1 < l_len)
            def _():
                drain(st1, mb1, semB)

                @pl.when(st1 >= 2)
                def _():
                    drain_out(st1 - 2, ob1, semO1)

                compute(l_start + st1, mb1, abuf, ob1)
                issue_out(st1, ob1, semO1)

            return carry

        lax.fori_loop(0, (MAXL + 1) // 2, t_body, 0)

        @pl.when(l_len == MAXL)
        def _():
            drain_out(MAXL - 1, ob0, semO0)
            drain_out(MAXL - 2, ob1, semO1)

        @pl.when(l_len == MAXL - 1)
        def _():
            drain_out(MAXL - 3, ob0, semO0)
            drain_out(MAXL - 2, ob1, semO1)

    return k(m_t, a_t)


def _tc_logits_t(m_t, l_t):
    BB = 128

    def body(m_ref, l_ref, o_ref, om_ref, ol_ref):
        l = l_ref[...]
        e = jnp.exp(l)
        e0 = e[:, 0:1, :]
        q = (1.0 - P) * e + PR * jnp.clip(1.0 - e - e0, 0.0, 1.0)
        col = lax.broadcasted_iota(jnp.int32, l.shape, 1)
        o_ref[...] = jnp.where(col == 0, l, jnp.log(q))
        om_ref[...] = m_ref[...]
        ol_ref[...] = l

    spec = pl.BlockSpec((L, V, BB), lambda i: (0, 0, i))
    return pl.pallas_call(
        body,
        grid=(B // BB,),
        in_specs=[spec, spec],
        out_specs=[spec, spec, spec],
        out_shape=[jax.ShapeDtypeStruct((L, V, B), jnp.float32)] * 3,
    )(m_t, l_t)


def kernel(messages, logits):
    m_t = jnp.transpose(messages, (1, 2, 0))
    l_t = jnp.transpose(logits, (1, 2, 0))
    ln_t, mcp_t, lcp_t = _tc_logits_t(m_t, l_t)
    mn2d = _sc_messages_t(m_t.reshape(R, B), jnp.asarray(_AT))
    mn = jnp.transpose(mn2d.reshape(L, V, B), (2, 0, 1))
    ln = jnp.transpose(ln_t, (2, 0, 1))
    return (mn, ln, jnp.transpose(mcp_t, (2, 0, 1)), jnp.transpose(lcp_t, (2, 0, 1)))


# speedup vs baseline: 5.8254x; 1.0277x over previous
import functools

import numpy as np
import jax
import jax.numpy as jnp
from jax import lax
from jax.experimental import pallas as pl
from jax.experimental.pallas import tpu as pltpu
from jax.experimental.pallas import tpu_sc as plsc

B, L, V = 1024, 50, 64
P = 0.05
N = B * L
R = L * V
NC, NS = 2, 16
NW = NC * NS
LB = 128
MAXL = 13
SCALE_S = 1.0 / (V - 2)
SCALE_G = float(V - 1) / (V - 2)
PR = float(P / (V - 2))


def _mask_t() -> np.ndarray:
    mask = np.random.RandomState(42).rand(N, V - 1) < P
    a = np.zeros((B, L, V), np.uint32)
    a[:, :, : V - 1] = mask.reshape(B, L, V - 1)
    at = np.ascontiguousarray(a.transpose(1, 2, 0))
    w = np.zeros((L, 2, B), np.uint32)
    for vv in range(32):
        w[:, 0, :] |= at[:, vv, :] << vv
        w[:, 1, :] |= at[:, vv + 32, :] << vv
    return w.reshape(L * 2, B).view(np.int32)


_AT = _mask_t()


def _sc_messages_t(m_t, a_t):
    mesh = plsc.VectorSubcoreMesh(core_axis_name="c", subcore_axis_name="s")

    def compute(l, mbuf, abuf, obuf):
        def group_body(g, carry):
            sl = pl.ds(g * 16, 16)
            one = jnp.ones((16,), jnp.int32)
            wlo = abuf[2 * l, sl]
            whi = abuf[2 * l + 1, sl]

            def bit(w, vv):
                return ((lax.shift_right_logical(w, jnp.full((16,), vv, jnp.int32)) & one)
                        .astype(jnp.float32))

            ghi = mbuf[31, sl] * bit(wlo, 31)
            glo = None
            s = jnp.zeros((16,), jnp.float32)
            for vv in range(32):
                mlo = mbuf[vv, sl]
                mhi = mbuf[vv + 32, sl]
                if vv == 0:
                    obuf[vv, sl] = mlo
                else:
                    obuf[vv, sl] = mlo - SCALE_G * glo
                obuf[vv + 32, sl] = mhi - SCALE_G * ghi
                glo = mlo * bit(wlo, vv)
                ghi = mhi * bit(whi, vv)
                s = s + glo + ghi
            sv = s * SCALE_S
            for v in range(1, V):
                plsc.addupdate(obuf.at[v, sl], sv)
            return carry

        lax.fori_loop(0, LB // 16, group_body, 0)

    @functools.partial(
        pl.kernel,
        out_type=jax.ShapeDtypeStruct((R, B), jnp.float32),
        mesh=mesh,
        scratch_types=[
            pltpu.VMEM((V, LB), jnp.float32),
            pltpu.VMEM((V, LB), jnp.float32),
            pltpu.VMEM((2 * L, LB), jnp.int32),
            pltpu.VMEM((V, LB), jnp.float32),
            pltpu.VMEM((V, LB), jnp.float32),
            pltpu.SemaphoreType.DMA,
            pltpu.SemaphoreType.DMA,
            pltpu.SemaphoreType.DMA,
            pltpu.SemaphoreType.DMA,
        ],
        compiler_params=pltpu.CompilerParams(needs_layout_passes=False),
    )
    def k(m_hbm, a_hbm, out_hbm, mb0, mb1, abuf, ob0, ob1, semA, semB, semO0, semO1):
        wid = lax.axis_index("s") * NC + lax.axis_index("c")
        lg = wid // 8
        b0 = (wid % 8) * LB
        l_start = MAXL * lg - jnp.maximum(lg - 2, 0)
        l_len = MAXL - (lg >= 2).astype(jnp.int32)

        def issue(st, mb, sem):
            l = l_start + st
            pltpu.async_copy(m_hbm.at[pl.ds(l * V, V), pl.ds(b0, LB)], mb, sem)

        def drain(st, mb, sem):
            l = l_start + st
            pltpu.make_async_copy(
                m_hbm.at[pl.ds(l * V, V), pl.ds(b0, LB)], mb, sem).wait()

        def issue_out(st, ob, sem):
            l = l_start + st
            pltpu.async_copy(ob, out_hbm.at[pl.ds(l * V, V), pl.ds(b0, LB)], sem)

        def drain_out(st, ob, sem):
            l = l_start + st
            pltpu.make_async_copy(
                ob, out_hbm.at[pl.ds(l * V, V), pl.ds(b0, LB)], sem).wait()

        ca = pltpu.async_copy(a_hbm.at[:, pl.ds(b0, LB)], abuf, semA)
        issue(0, mb0, semA)
        ca.wait()

        def t_body(t, carry):
            st0 = 2 * t
            st1 = st0 + 1

            @pl.when(st1 < l_len)
            def _():
                issue(st1, mb1, semB)

            @pl.when(st0 < l_len)
            def _():
                drain(st0, mb0, semA)

                @pl.when(st0 >= 2)
                def _():
                    drain_out(st0 - 2, ob0, semO0)

                compute(l_start + st0, mb0, abuf, ob0)
                issue_out(st0, ob0, semO0)

            @pl.when(st0 + 2 < l_len)
            def _():
                issue(st0 + 2, mb0, semA)

            @pl.when(st1 < l_len)
            def _():
                drain(st1, mb1, semB)

                @pl.when(st1 >= 2)
                def _():
                    drain_out(st1 - 2, ob1, semO1)

                compute(l_start + st1, mb1, abuf, ob1)
                issue_out(st1, ob1, semO1)

            return carry

        lax.fori_loop(0, (MAXL + 1) // 2, t_body, 0)

        @pl.when(l_len == MAXL)
        def _():
            drain_out(MAXL - 1, ob0, semO0)
            drain_out(MAXL - 2, ob1, semO1)

        @pl.when(l_len == MAXL - 1)
        def _():
            drain_out(MAXL - 3, ob0, semO0)
            drain_out(MAXL - 2, ob1, semO1)

    return k(m_t, a_t)


def _tc_logits_t(m_t, l_t):
    BB = 256

    def body(m_ref, l_ref, o_ref, om_ref, ol_ref):
        l = l_ref[...]
        e = jnp.exp(l)
        e0 = e[:, 0:1, :]
        q = (1.0 - P) * e + PR * jnp.clip(1.0 - e - e0, 0.0, 1.0)
        col = lax.broadcasted_iota(jnp.int32, l.shape, 1)
        o_ref[...] = jnp.where(col == 0, l, jnp.log(q))
        om_ref[...] = m_ref[...]
        ol_ref[...] = l

    spec = pl.BlockSpec((L, V, BB), lambda i: (0, 0, i))
    return pl.pallas_call(
        body,
        grid=(B // BB,),
        in_specs=[spec, spec],
        out_specs=[spec, spec, spec],
        out_shape=[jax.ShapeDtypeStruct((L, V, B), jnp.float32)] * 3,
    )(m_t, l_t)


def kernel(messages, logits):
    m_t = jnp.transpose(messages, (1, 2, 0))
    l_t = jnp.transpose(logits, (1, 2, 0))
    ln_t, mcp_t, lcp_t = _tc_logits_t(m_t, l_t)
    mn2d = _sc_messages_t(m_t.reshape(R, B), jnp.asarray(_AT))
    mn = jnp.transpose(mn2d.reshape(L, V, B), (2, 0, 1))
    ln = jnp.transpose(ln_t, (2, 0, 1))
    return (mn, ln, jnp.transpose(mcp_t, (2, 0, 1)), jnp.transpose(lcp_t, (2, 0, 1)))
